# baseline (speedup 1.0000x reference)
import functools

import jax
import jax.numpy as jnp
from jax import lax
from jax.experimental import pallas as pl
from jax.experimental.pallas import tpu as pltpu
from jax.experimental.pallas import tpu_sc as plsc

_NC = 2
_NS = 16
_NW = _NC * _NS
_L = 16

_N = 128 * 32768
_PER_W = _N // _NW
_CHUNK = 16384
_NCHUNK = _PER_W // _CHUNK
_SLICES = _CHUNK // _L

_IMAX = 2**31 - 1


def _sc_partial_argmin(x_hbm, pval_hbm, pidx_hbm, buf, pv, pi):
    wid = lax.axis_index("s") * _NC + lax.axis_index("c")
    base = wid * _PER_W
    iota = lax.iota(jnp.int32, _L)

    minv = jnp.full((_L,), jnp.inf, jnp.float32)
    mini = jnp.zeros((_L,), jnp.int32)

    for c in range(_NCHUNK):
        off = base + c * _CHUNK
        pltpu.sync_copy(x_hbm.at[pl.ds(off, _CHUNK)], buf)
        chunk_base = off

        def body(i, carry, chunk_base=chunk_base):
            mv, mi = carry
            v = buf[pl.ds(i * _L, _L)]
            idx = chunk_base + i * _L + iota
            take = v < mv
            mv = jnp.where(take, v, mv)
            mi = jnp.where(take, idx, mi)
            return mv, mi

        minv, mini = lax.fori_loop(0, _SLICES, body, (minv, mini))

    pv[...] = minv
    pi[...] = mini
    pltpu.sync_copy(pv, pval_hbm.at[wid])
    pltpu.sync_copy(pi, pidx_hbm.at[wid])


def _merge_body(pval_ref, pidx_ref, out_ref):
    vals = pval_ref[...]
    idxs = pidx_ref[...]
    m = jnp.min(vals)
    out_ref[0, 0] = jnp.min(jnp.where(vals == m, idxs, jnp.int32(_IMAX)))


def kernel(x):
    xf = x.reshape(-1)

    sc = functools.partial(
        pl.kernel,
        out_type=[
            jax.ShapeDtypeStruct((_NW, _L), jnp.float32),
            jax.ShapeDtypeStruct((_NW, _L), jnp.int32),
        ],
        mesh=plsc.VectorSubcoreMesh(core_axis_name="c", subcore_axis_name="s"),
        scratch_types=[
            pltpu.VMEM((_CHUNK,), jnp.float32),
            pltpu.VMEM((_L,), jnp.float32),
            pltpu.VMEM((_L,), jnp.int32),
        ],
    )(_sc_partial_argmin)
    pvals, pidxs = sc(xf)

    out = pl.pallas_call(
        _merge_body,
        out_shape=jax.ShapeDtypeStruct((1, 1), jnp.int32),
        out_specs=pl.BlockSpec(memory_space=pltpu.SMEM),
    )(pvals, pidxs)
    return out


# speedup vs baseline: 2.5146x; 2.5146x over previous
import functools

import jax
import jax.numpy as jnp
from jax import lax
from jax.experimental import pallas as pl
from jax.experimental.pallas import tpu as pltpu
from jax.experimental.pallas import tpu_sc as plsc

_NC = 2
_NS = 16
_NW = _NC * _NS
_L = 16

_R = 128
_C = 32768
_ROWS_W = _R // _NW
_PER_W = _ROWS_W * _C
_CHUNK = 16384
_CPR = _C // _CHUNK
_NCHUNK = _PER_W // _CHUNK
_U = 8
_ITERS = _CHUNK // (_U * _L)

_IMAX = 2**31 - 1


def _sc_partial_argmin(x_hbm, pval_hbm, pidx_hbm, buf, pv, pi, sem0, sem1):
    wid = lax.axis_index("s") * _NC + lax.axis_index("c")
    row0 = wid * _ROWS_W
    bufs = [buf.at[0], buf.at[1]]
    sems = [sem0, sem1]

    def start(c):
        r = row0 + (c // _CPR)
        coff = (c % _CPR) * _CHUNK
        return pltpu.async_copy(
            x_hbm.at[r, pl.ds(coff, _CHUNK)], bufs[c % 2], sems[c % 2]
        )

    mvs = [jnp.full((_L,), jnp.inf, jnp.float32) for _ in range(_U)]
    mss = [jnp.zeros((_L,), jnp.int32) for _ in range(_U)]
    gvec = jnp.zeros((_L,), jnp.int32)

    pending = start(0)
    for c in range(_NCHUNK):
        pending.wait()
        if c + 1 < _NCHUNK:
            pending = start(c + 1)
        cbuf = bufs[c % 2]

        def body(i, carry, cbuf=cbuf):
            mvs = list(carry[:_U])
            mss = list(carry[_U:2 * _U])
            gv = carry[2 * _U]
            base = i * (_U * _L)
            for k in range(_U):
                v = cbuf[pl.ds(base + k * _L, _L)]
                take = v < mvs[k]
                mvs[k] = jnp.minimum(v, mvs[k])
                mss[k] = jnp.where(take, gv, mss[k])
            return (*mvs, *mss, gv + 1)

        carry = lax.fori_loop(0, _ITERS, body, (*mvs, *mss, gvec))
        mvs = list(carry[:_U])
        mss = list(carry[_U:2 * _U])
        gvec = carry[2 * _U]

    for k in range(_U):
        pv[k] = mvs[k]
        pi[k] = mss[k]
    pltpu.sync_copy(pv, pval_hbm.at[wid])
    pltpu.sync_copy(pi, pidx_hbm.at[wid])


def _merge_body(pval_ref, pidx_ref, out_ref):
    vals = pval_ref[...]
    its = pidx_ref[...]
    shape = (_NW, _U, _L)
    wid = lax.broadcasted_iota(jnp.int32, shape, 0)
    k = lax.broadcasted_iota(jnp.int32, shape, 1)
    lane = lax.broadcasted_iota(jnp.int32, shape, 2)
    idx = wid * _PER_W + (its * _U + k) * _L + lane
    m = jnp.min(vals)
    out_ref[0, 0] = jnp.min(jnp.where(vals == m, idx, jnp.int32(_IMAX)))


def kernel(x):
    sc = functools.partial(
        pl.kernel,
        out_type=[
            jax.ShapeDtypeStruct((_NW, _U, _L), jnp.float32),
            jax.ShapeDtypeStruct((_NW, _U, _L), jnp.int32),
        ],
        mesh=plsc.VectorSubcoreMesh(core_axis_name="c", subcore_axis_name="s"),
        scratch_types=[
            pltpu.VMEM((2, _CHUNK), jnp.float32),
            pltpu.VMEM((_U, _L), jnp.float32),
            pltpu.VMEM((_U, _L), jnp.int32),
            pltpu.SemaphoreType.DMA,
            pltpu.SemaphoreType.DMA,
        ],
    )(_sc_partial_argmin)
    pvals, pidxs = sc(x)

    out = pl.pallas_call(
        _merge_body,
        out_shape=jax.ShapeDtypeStruct((1, 1), jnp.int32),
        out_specs=pl.BlockSpec(memory_space=pltpu.SMEM),
    )(pvals, pidxs)
    return out


# speedup vs baseline: 2.5893x; 1.0297x over previous
import functools

import jax
import jax.numpy as jnp
from jax import lax
from jax.experimental import pallas as pl
from jax.experimental.pallas import tpu as pltpu
from jax.experimental.pallas import tpu_sc as plsc

_NC = 2
_NS = 16
_NW = _NC * _NS
_L = 16

_R = 128
_C = 32768
_ROWS_W = _R // _NW
_PER_W = _ROWS_W * _C
_CHUNK = 32768
_CPR = _C // _CHUNK
_NCHUNK = _PER_W // _CHUNK
_U = 16
_ITERS = _CHUNK // (_U * _L)

_IMAX = 2**31 - 1


def _sc_partial_argmin(x_hbm, pval_hbm, pidx_hbm, buf, pv, pi, sem0, sem1):
    wid = lax.axis_index("s") * _NC + lax.axis_index("c")
    row0 = wid * _ROWS_W
    bufs = [buf.at[0], buf.at[1]]
    sems = [sem0, sem1]

    def start(c):
        r = row0 + (c // _CPR)
        coff = (c % _CPR) * _CHUNK
        return pltpu.async_copy(
            x_hbm.at[r, pl.ds(coff, _CHUNK)], bufs[c % 2], sems[c % 2]
        )

    mvs = [jnp.full((_L,), jnp.inf, jnp.float32) for _ in range(_U)]
    mss = [jnp.zeros((_L,), jnp.int32) for _ in range(_U)]
    gvec = jnp.zeros((_L,), jnp.int32)

    pending = start(0)
    for c in range(_NCHUNK):
        pending.wait()
        if c + 1 < _NCHUNK:
            pending = start(c + 1)
        cbuf = bufs[c % 2]

        def body(i, carry, cbuf=cbuf):
            mvs = list(carry[:_U])
            mss = list(carry[_U:2 * _U])
            gv = carry[2 * _U]
            base = i * (_U * _L)
            for k in range(_U):
                v = cbuf[pl.ds(base + k * _L, _L)]
                take = v < mvs[k]
                mvs[k] = jnp.minimum(v, mvs[k])
                mss[k] = jnp.where(take, gv, mss[k])
            return (*mvs, *mss, gv + 1)

        carry = lax.fori_loop(0, _ITERS, body, (*mvs, *mss, gvec))
        mvs = list(carry[:_U])
        mss = list(carry[_U:2 * _U])
        gvec = carry[2 * _U]

    for k in range(_U):
        pv[k] = mvs[k]
        pi[k] = mss[k]
    pltpu.sync_copy(pv, pval_hbm.at[wid])
    pltpu.sync_copy(pi, pidx_hbm.at[wid])


def _merge_body(pval_ref, pidx_ref, out_ref):
    vals = pval_ref[...]
    its = pidx_ref[...]
    shape = (_NW, _U, _L)
    wid = lax.broadcasted_iota(jnp.int32, shape, 0)
    k = lax.broadcasted_iota(jnp.int32, shape, 1)
    lane = lax.broadcasted_iota(jnp.int32, shape, 2)
    idx = wid * _PER_W + (its * _U + k) * _L + lane
    m = jnp.min(vals)
    out_ref[0, 0] = jnp.min(jnp.where(vals == m, idx, jnp.int32(_IMAX)))


def kernel(x):
    sc = functools.partial(
        pl.kernel,
        out_type=[
            jax.ShapeDtypeStruct((_NW, _U, _L), jnp.float32),
            jax.ShapeDtypeStruct((_NW, _U, _L), jnp.int32),
        ],
        mesh=plsc.VectorSubcoreMesh(core_axis_name="c", subcore_axis_name="s"),
        scratch_types=[
            pltpu.VMEM((2, _CHUNK), jnp.float32),
            pltpu.VMEM((_U, _L), jnp.float32),
            pltpu.VMEM((_U, _L), jnp.int32),
            pltpu.SemaphoreType.DMA,
            pltpu.SemaphoreType.DMA,
        ],
    )(_sc_partial_argmin)
    pvals, pidxs = sc(x)

    out = pl.pallas_call(
        _merge_body,
        out_shape=jax.ShapeDtypeStruct((1, 1), jnp.int32),
        out_specs=pl.BlockSpec(memory_space=pltpu.SMEM),
    )(pvals, pidxs)
    return out
